# Initial kernel scaffold; baseline (speedup 1.0000x reference)
import functools

import jax
import jax.numpy as jnp
from jax import lax
from jax.experimental import pallas as pl
from jax.experimental.pallas import tpu as pltpu
from jax.experimental.pallas import tpu_sc as plsc


def _sc_threshold(xflat, idxT, wT, bias, *, B, IN, R, T):
    info = plsc.get_sparse_core_info()
    NC, NS = info.num_cores, info.num_subcores
    NW = NC * NS
    B_CH = 16
    N_CH = B // (NW * B_CH)
    R_CH = 512
    N_RC = R // R_CH

    mesh = plsc.VectorSubcoreMesh(core_axis_name="c", subcore_axis_name="s")

    @functools.partial(
        pl.kernel,
        mesh=mesh,
        out_type=jax.ShapeDtypeStruct((B, R), jnp.float32),
        scratch_types=[
            pltpu.VMEM((B_CH * IN,), jnp.float32),
            pltpu.VMEM((T, R_CH), jnp.int32),
            pltpu.VMEM((T, R_CH), jnp.float32),
            pltpu.VMEM((R_CH,), jnp.float32),
            pltpu.VMEM((B_CH, R_CH), jnp.float32),
        ],
    )
    def k(xf_hbm, idxT_hbm, wT_hbm, bias_hbm, y_hbm, xv, iv, wv, bv, yv):
        wid = lax.axis_index("s") * NC + lax.axis_index("c")

        def cb_body(cb, carry):
            b0 = (wid * N_CH + cb) * B_CH
            pltpu.sync_copy(xf_hbm.at[pl.ds(b0 * IN, B_CH * IN)], xv)

            def rc_body(rc, carry):
                r0 = rc * R_CH
                pltpu.sync_copy(idxT_hbm.at[:, pl.ds(r0, R_CH)], iv)
                pltpu.sync_copy(wT_hbm.at[:, pl.ds(r0, R_CH)], wv)
                pltpu.sync_copy(bias_hbm.at[pl.ds(r0, R_CH)], bv)

                def g_body(g, carry):
                    goff = g * 16
                    idxg = [iv[t, pl.ds(goff, 16)] for t in range(T)]
                    wg = [wv[t, pl.ds(goff, 16)] for t in range(T)]
                    nb = -bv[pl.ds(goff, 16)]
                    for b in range(B_CH):
                        acc = nb
                        ofs = b * IN
                        for t in range(T):
                            v = plsc.load_gather(xv, [idxg[t] + ofs])
                            acc = acc + wg[t] * v
                        yv[b, pl.ds(goff, 16)] = acc
                    return carry

                lax.fori_loop(0, R_CH // 16, g_body, carry)
                pltpu.sync_copy(yv, y_hbm.at[pl.ds(b0, B_CH), pl.ds(r0, R_CH)])
                return carry

            return lax.fori_loop(0, N_RC, rc_body, carry)

        lax.fori_loop(0, N_CH, cb_body, 0)

    return k(xflat, idxT, wT, bias)


def kernel(x, weight_k, bias, idx):
    B, IN = x.shape
    R, T = idx.shape
    xflat = x.reshape(-1)
    idxT = idx.T
    wT = weight_k.T
    return _sc_threshold(xflat, idxT, wT, bias, B=B, IN=IN, R=R, T=T)


# speedup vs baseline: 1.3746x; 1.3746x over previous
import functools

import jax
import jax.numpy as jnp
from jax import lax
from jax.experimental import pallas as pl
from jax.experimental.pallas import tpu as pltpu
from jax.experimental.pallas import tpu_sc as plsc


def _sc_threshold(xflat, idxT, wT, bias, *, B, IN, R, T):
    info = plsc.get_sparse_core_info()
    NC, NS = info.num_cores, info.num_subcores
    NW = NC * NS
    B_CH = 16
    N_CH = B // (NW * B_CH)
    R_CH = 512
    N_RC = R // R_CH

    mesh = plsc.VectorSubcoreMesh(core_axis_name="c", subcore_axis_name="s")

    @functools.partial(
        pl.kernel,
        mesh=mesh,
        compiler_params=pltpu.CompilerParams(needs_layout_passes=False),
        out_type=jax.ShapeDtypeStruct((B, R), jnp.float32),
        scratch_types=[
            pltpu.VMEM((B_CH * IN,), jnp.float32),
            pltpu.VMEM((T, R_CH), jnp.int32),
            pltpu.VMEM((T, R_CH), jnp.float32),
            pltpu.VMEM((R_CH,), jnp.float32),
            pltpu.VMEM((B_CH, R_CH), jnp.float32),
        ],
    )
    def k(xf_hbm, idxT_hbm, wT_hbm, bias_hbm, y_hbm, xv, iv, wv, bv, yv):
        wid = lax.axis_index("s") * NC + lax.axis_index("c")

        def cb_body(cb, carry):
            b0 = (wid * N_CH + cb) * B_CH
            pltpu.sync_copy(xf_hbm.at[pl.ds(b0 * IN, B_CH * IN)], xv)

            def rc_body(rc, carry):
                r0 = rc * R_CH
                pltpu.sync_copy(idxT_hbm.at[:, pl.ds(r0, R_CH)], iv)
                pltpu.sync_copy(wT_hbm.at[:, pl.ds(r0, R_CH)], wv)
                pltpu.sync_copy(bias_hbm.at[pl.ds(r0, R_CH)], bv)

                def g_body(g, carry):
                    goff = g * 16
                    idxg = [iv[t, pl.ds(goff, 16)] for t in range(T)]
                    wg = [wv[t, pl.ds(goff, 16)] for t in range(T)]
                    nb = -bv[pl.ds(goff, 16)]
                    for b in range(B_CH):
                        acc = nb
                        ofs = b * IN
                        for t in range(T):
                            v = plsc.load_gather(xv, [idxg[t] + ofs])
                            acc = acc + wg[t] * v
                        yv[b, pl.ds(goff, 16)] = acc
                    return carry

                lax.fori_loop(0, R_CH // 16, g_body, carry)
                pltpu.sync_copy(yv, y_hbm.at[pl.ds(b0, B_CH), pl.ds(r0, R_CH)])
                return carry

            return lax.fori_loop(0, N_RC, rc_body, carry)

        lax.fori_loop(0, N_CH, cb_body, 0)

    return k(xflat, idxT, wT, bias)


def kernel(x, weight_k, bias, idx):
    B, IN = x.shape
    R, T = idx.shape
    xflat = x.reshape(-1)
    idxT = idx.T
    wT = weight_k.T
    return _sc_threshold(xflat, idxT, wT, bias, B=B, IN=IN, R=R, T=T)
